# baseline (speedup 1.0000x reference)
import functools

import jax
import jax.numpy as jnp
from jax import lax
from jax.experimental import pallas as pl
from jax.experimental.pallas import tpu as pltpu
from jax.experimental.pallas import tpu_sc as plsc

N_NODES = 100000
N_EDGES = 6400000
N_GRAPHS = 256
IN_CH = 12
HID = 64
OUT_CH = 4

NC, NS = 2, 16
CHUNK = 128
STG = 4
DEPTH = 3
N_PAD = 100096
E_PAD = 6553600
NCHUNKS = E_PAD // CHUNK
ROWS_PER_TILE = N_PAD // NS

B = 2000
NB = N_NODES // B

_mesh = plsc.VectorSubcoreMesh(
    core_axis_name="c", subcore_axis_name="s", num_cores=NC, num_subcores=NS)


def _f32(shape):
  return jax.ShapeDtypeStruct(shape, jnp.float32)


def _edge_pipeline(ngroups, tile_c0, src_hbm, dst_hbm, tbl_hbm,
                   idxs, idxd, rows, acc, semst, semg, sems):
  gather = tbl_hbm is not None
  lag = DEPTH - 1

  def stage(k, b):
    c0 = tile_c0 + k * STG
    if gather:
      pltpu.async_copy(src_hbm.at[pl.ds(c0, STG)], idxs.at[b], semst)
    pltpu.async_copy(dst_hbm.at[pl.ds(c0, STG)], idxd.at[b], semst)

  def drain_stage(b):
    if gather:
      pltpu.make_async_copy(src_hbm.at[pl.ds(0, STG)], idxs.at[b],
                            semst).wait()
    pltpu.make_async_copy(dst_hbm.at[pl.ds(0, STG)], idxd.at[b],
                          semst).wait()

  def drain_scatters(b):
    def drain_s(j, c2):
      if gather:
        pltpu.make_async_copy(rows.at[b, j], acc.at[idxd.at[b, j]],
                              sems).wait()
      else:
        pltpu.make_async_copy(rows, acc.at[idxd.at[b, j]], sems).wait()
      return c2

    lax.fori_loop(0, STG, drain_s, 0)

  stage(0, 0)

  def group_body(k, carry):
    b = lax.rem(k, DEPTH)

    @pl.when(k >= lag)
    def _():
      drain_scatters(lax.rem(k + 1, DEPTH))

    drain_stage(b)

    @pl.when(k + 1 < ngroups)
    def _():
      stage(k + 1, lax.rem(k + 1, DEPTH))

    if gather:
      def fire_g(j, c2):
        pltpu.async_copy(tbl_hbm.at[idxs.at[b, j]], rows.at[b, j], semg)
        return c2

      lax.fori_loop(0, STG, fire_g, 0)

      def drain_g(j, c2):
        pltpu.make_async_copy(tbl_hbm.at[idxs.at[b, j]], rows.at[b, j],
                              semg).wait()
        return c2

      lax.fori_loop(0, STG, drain_g, 0)

    def fire_s(j, c2):
      if gather:
        pltpu.async_copy(rows.at[b, j], acc.at[idxd.at[b, j]], sems,
                         add=True)
      else:
        pltpu.async_copy(rows, acc.at[idxd.at[b, j]], sems, add=True)
      return c2

    lax.fori_loop(0, STG, fire_s, 0)
    return carry

  lax.fori_loop(0, ngroups, group_body, 0)

  def drain_tail(k, carry):
    drain_scatters(lax.rem(k, DEPTH))
    return carry

  lax.fori_loop(jnp.maximum(ngroups - lag, 0), ngroups, drain_tail, 0)


@functools.partial(
    pl.kernel,
    out_type=[_f32((N_PAD, 16)), _f32((N_PAD, 16))],
    mesh=_mesh,
    compiler_params=pltpu.CompilerParams(use_tc_tiling_on_sc=False),
    scratch_types=[
        pltpu.VMEM((DEPTH, STG, CHUNK), jnp.int32),
        pltpu.VMEM((CHUNK, 16), jnp.float32),
        pltpu.VMEM_SHARED((N_PAD, 16), jnp.float32),
        pltpu.SemaphoreType.DMA,
        pltpu.SemaphoreType.DMA,
    ],
)
def _deg_kernel(dst_hbm, ones_hbm, zeros_hbm, out0, out1,
                idxd, ones_v, acc, semst, sems):
  c = lax.axis_index("c")
  s = lax.axis_index("s")
  w = c * NS + s

  pltpu.sync_copy(ones_hbm, ones_v)
  row0 = s * ROWS_PER_TILE
  pltpu.sync_copy(zeros_hbm.at[pl.ds(row0, ROWS_PER_TILE)],
                  acc.at[pl.ds(row0, ROWS_PER_TILE)])
  plsc.subcore_barrier()

  chunks_per_tile = NCHUNKS // (NC * NS)
  _edge_pipeline(chunks_per_tile // STG, w * chunks_per_tile,
                 None, dst_hbm, None, None, idxd, ones_v, acc,
                 semst, None, sems)
  plsc.subcore_barrier()

  @pl.when(c == 0)
  def _():
    pltpu.sync_copy(acc.at[pl.ds(row0, ROWS_PER_TILE)],
                    out0.at[pl.ds(row0, ROWS_PER_TILE)])

  @pl.when(c == 1)
  def _():
    pltpu.sync_copy(acc.at[pl.ds(row0, ROWS_PER_TILE)],
                    out1.at[pl.ds(row0, ROWS_PER_TILE)])


@functools.partial(
    pl.kernel,
    out_type=[_f32((N_PAD, 16)), _f32((N_PAD, 16))],
    mesh=_mesh,
    compiler_params=pltpu.CompilerParams(use_tc_tiling_on_sc=False),
    scratch_types=[
        pltpu.VMEM((DEPTH, STG, CHUNK), jnp.int32),
        pltpu.VMEM((DEPTH, STG, CHUNK), jnp.int32),
        pltpu.VMEM((DEPTH, STG, CHUNK, 16), jnp.float32),
        pltpu.VMEM_SHARED((N_PAD, 16), jnp.float32),
        pltpu.SemaphoreType.DMA,
        pltpu.SemaphoreType.DMA,
        pltpu.SemaphoreType.DMA,
    ],
)
def _agg1_kernel(src_hbm, dst_hbm, tbl_hbm, zeros_hbm, out0, out1,
                 idxs, idxd, rows, acc, semst, semg, sems):
  c = lax.axis_index("c")
  s = lax.axis_index("s")

  row0 = s * ROWS_PER_TILE
  pltpu.sync_copy(zeros_hbm.at[pl.ds(row0, ROWS_PER_TILE)],
                  acc.at[pl.ds(row0, ROWS_PER_TILE)])
  plsc.subcore_barrier()

  chunks_per_tile = (NCHUNKS // NC) // NS
  tile_c0 = c * (NCHUNKS // NC) + s * chunks_per_tile
  _edge_pipeline(chunks_per_tile // STG, tile_c0,
                 src_hbm, dst_hbm, tbl_hbm, idxs, idxd, rows, acc,
                 semst, semg, sems)
  plsc.subcore_barrier()

  @pl.when(c == 0)
  def _():
    pltpu.sync_copy(acc.at[pl.ds(row0, ROWS_PER_TILE)],
                    out0.at[pl.ds(row0, ROWS_PER_TILE)])

  @pl.when(c == 1)
  def _():
    pltpu.sync_copy(acc.at[pl.ds(row0, ROWS_PER_TILE)],
                    out1.at[pl.ds(row0, ROWS_PER_TILE)])


@functools.partial(
    pl.kernel,
    out_type=[_f32((N_PAD, 16))] * 4,
    mesh=_mesh,
    compiler_params=pltpu.CompilerParams(use_tc_tiling_on_sc=False),
    scratch_types=[
        pltpu.VMEM((DEPTH, STG, CHUNK), jnp.int32),
        pltpu.VMEM((DEPTH, STG, CHUNK), jnp.int32),
        pltpu.VMEM((DEPTH, STG, CHUNK, 16), jnp.float32),
        pltpu.VMEM_SHARED((N_PAD, 16), jnp.float32),
        pltpu.SemaphoreType.DMA,
        pltpu.SemaphoreType.DMA,
        pltpu.SemaphoreType.DMA,
    ],
)
def _agg2_kernel(src_hbm, dst_hbm, h0, h1, h2, h3, zeros_hbm,
                 o0, o1, o2, o3, idxs, idxd, rows, acc, semst, semg, sems):
  c = lax.axis_index("c")
  s = lax.axis_index("s")
  tbls = [h0, h1, h2, h3]
  outs = [o0, o1, o2, o3]

  row0 = s * ROWS_PER_TILE
  chunks_per_tile = NCHUNKS // NS

  for g in range(4):
    active = c == (g // 2)

    @pl.when(active)
    def _(g=g):
      pltpu.sync_copy(zeros_hbm.at[pl.ds(row0, ROWS_PER_TILE)],
                      acc.at[pl.ds(row0, ROWS_PER_TILE)])

    plsc.subcore_barrier()

    @pl.when(active)
    def _(g=g):
      _edge_pipeline(chunks_per_tile // STG, s * chunks_per_tile,
                     src_hbm, dst_hbm, tbls[g], idxs, idxd, rows, acc,
                     semst, semg, sems)

    plsc.subcore_barrier()

    @pl.when(active)
    def _(g=g):
      pltpu.sync_copy(acc.at[pl.ds(row0, ROWS_PER_TILE)],
                      outs[g].at[pl.ds(row0, ROWS_PER_TILE)])

    plsc.subcore_barrier()


def _scale_body(x_ref, d0_ref, d1_ref, p_ref, xs_ref, dinv_ref):
  deg = d0_ref[:, 0:1] + d1_ref[:, 0:1] + 1.0
  dinv = lax.rsqrt(deg)
  xs_ref[...] = (x_ref[...] * dinv) @ p_ref[...]
  dinv_ref[...] = dinv


def _scale_call(x, degp0, degp1, pmat):
  return pl.pallas_call(
      _scale_body,
      grid=(NB,),
      in_specs=[
          pl.BlockSpec((B, IN_CH), lambda i: (i, 0)),
          pl.BlockSpec((B, 16), lambda i: (i, 0)),
          pl.BlockSpec((B, 16), lambda i: (i, 0)),
          pl.BlockSpec((IN_CH, 16), lambda i: (0, 0)),
      ],
      out_specs=[
          pl.BlockSpec((B, 16), lambda i: (i, 0)),
          pl.BlockSpec((B, 1), lambda i: (i, 0)),
      ],
      out_shape=[_f32((N_PAD, 16)), _f32((N_NODES, 1))],
  )(x, degp0, degp1, pmat)


def _h1_body(p0_ref, p1_ref, xs_ref, dinv_ref, w1_ref, b1_ref,
             h0_ref, h1_ref, h2_ref, h3_ref):
  dinv = dinv_ref[...]
  t = (p0_ref[...] + p1_ref[...] + xs_ref[...]) * dinv
  outs = [h0_ref, h1_ref, h2_ref, h3_ref]
  for g in range(4):
    hg = jnp.maximum(t @ w1_ref[g] + b1_ref[g], 0.0)
    outs[g][...] = hg * dinv


def _h1_call(p0, p1, xs16, dinv, w1s, b1r):
  return pl.pallas_call(
      _h1_body,
      grid=(NB,),
      in_specs=[
          pl.BlockSpec((B, 16), lambda i: (i, 0)),
          pl.BlockSpec((B, 16), lambda i: (i, 0)),
          pl.BlockSpec((B, 16), lambda i: (i, 0)),
          pl.BlockSpec((B, 1), lambda i: (i, 0)),
          pl.BlockSpec((4, 16, 16), lambda i: (0, 0, 0)),
          pl.BlockSpec((4, 16), lambda i: (0, 0)),
      ],
      out_specs=[pl.BlockSpec((B, 16), lambda i: (i, 0))] * 4,
      out_shape=[_f32((N_PAD, 16))] * 4,
  )(p0, p1, xs16, dinv, w1s, b1r)


def _pool_body(o0_ref, o1_ref, o2_ref, o3_ref,
               s0_ref, s1_ref, s2_ref, s3_ref,
               dinv_ref, batch_ref, b2_ref, wfc_ref, bfc_ref,
               out_ref, sums_ref, cnt_ref):
  i = pl.program_id(0)

  @pl.when(i == 0)
  def _():
    sums_ref[...] = jnp.zeros((4, N_GRAPHS, 16), jnp.float32)
    cnt_ref[...] = jnp.zeros((N_GRAPHS, 1), jnp.float32)

  dinv = dinv_ref[...]
  bb = batch_ref[0]
  oht = (lax.broadcasted_iota(jnp.int32, (N_GRAPHS, B), 0) == bb
         ).astype(jnp.float32)
  cnt_ref[...] += lax.dot_general(
      oht, jnp.ones((B, 1), jnp.float32), (((1,), (0,)), ((), ())),
      preferred_element_type=jnp.float32)
  os_ = [o0_ref, o1_ref, o2_ref, o3_ref]
  ss_ = [s0_ref, s1_ref, s2_ref, s3_ref]
  for g in range(4):
    h2g = jnp.maximum((os_[g][...] + ss_[g][...]) * dinv + b2_ref[g], 0.0)
    sums_ref[g] += lax.dot_general(
        oht, h2g, (((1,), (0,)), ((), ())),
        preferred_element_type=jnp.float32)

  @pl.when(i == NB - 1)
  def _():
    cnt = jnp.maximum(cnt_ref[...], 1.0)
    logits = bfc_ref[...]
    for g in range(4):
      logits = logits + lax.dot_general(
          sums_ref[g] / cnt, wfc_ref[g], (((1,), (0,)), ((), ())),
          preferred_element_type=jnp.float32)
    out_ref[...] = 1.0 / (1.0 + jnp.exp(-logits))


def _pool_call(o_parts, h1s_parts, dinv, batch2d, b2r, wfcs, bfc2d):
  return pl.pallas_call(
      _pool_body,
      grid=(NB,),
      in_specs=(
          [pl.BlockSpec((B, 16), lambda i: (i, 0))] * 8 + [
              pl.BlockSpec((B, 1), lambda i: (i, 0)),
              pl.BlockSpec((1, 1, B), lambda i: (i, 0, 0)),
              pl.BlockSpec((4, 16), lambda i: (0, 0)),
              pl.BlockSpec((4, 16, OUT_CH), lambda i: (0, 0, 0)),
              pl.BlockSpec((1, OUT_CH), lambda i: (0, 0)),
          ]),
      out_specs=pl.BlockSpec((N_GRAPHS, OUT_CH), lambda i: (0, 0)),
      out_shape=_f32((N_GRAPHS, OUT_CH)),
      scratch_shapes=[
          pltpu.VMEM((4, N_GRAPHS, 16), jnp.float32),
          pltpu.VMEM((N_GRAPHS, 1), jnp.float32),
      ],
  )(*o_parts, *h1s_parts, dinv, batch2d, b2r, wfcs, bfc2d)


def kernel(x, edge_index, batch, W1, b1, W2, b2, Wfc, bfc):
  f32 = jnp.float32
  npad_e = E_PAD - N_EDGES
  dummy = jnp.full((npad_e,), N_NODES, jnp.int32)
  src2d = jnp.concatenate([edge_index[0], dummy]).reshape(NCHUNKS, CHUNK)
  dst2d = jnp.concatenate([edge_index[1], dummy]).reshape(NCHUNKS, CHUNK)

  ones_rows = jnp.ones((CHUNK, 16), f32)
  zeros_tbl = jnp.zeros((N_PAD, 16), f32)
  pmat = jnp.eye(IN_CH, 16, dtype=f32)

  w1p = jnp.concatenate([W1, jnp.zeros((16 - IN_CH, HID), f32)], axis=0)
  w1s = jnp.stack([w1p[:, 16 * g:16 * (g + 1)] for g in range(4)])
  b1r = b1.reshape(4, 16)
  b2r = b2.reshape(4, 16)
  wfcs = jnp.stack([Wfc[16 * g:16 * (g + 1), :] for g in range(4)])
  bfc2d = bfc.reshape(1, OUT_CH)
  batch2d = batch.reshape(NB, 1, B)

  degp0, degp1 = _deg_kernel(dst2d, ones_rows, zeros_tbl)
  xs16, dinv = _scale_call(x, degp0, degp1, pmat)
  p0, p1 = _agg1_kernel(src2d, dst2d, xs16, zeros_tbl)
  h1s = _h1_call(p0, p1, xs16, dinv, w1s, b1r)
  o_parts = _agg2_kernel(src2d, dst2d, *h1s, zeros_tbl)
  return _pool_call(o_parts, h1s, dinv, batch2d, b2r, wfcs, bfc2d)


# speedup vs baseline: 1.7639x; 1.7639x over previous
import functools

import jax
import jax.numpy as jnp
from jax import lax
from jax.experimental import pallas as pl
from jax.experimental.pallas import tpu as pltpu
from jax.experimental.pallas import tpu_sc as plsc

N_NODES = 100000
N_EDGES = 6400000
N_GRAPHS = 256
IN_CH = 12
HID = 64
OUT_CH = 4

NC, NS = 2, 16
CHUNK = 128
STG = 4
DEPTH = 3
N_PAD = 100096
E_PAD = 6406144
NCHUNKS = E_PAD // CHUNK
ROWS_PER_TILE = N_PAD // NS

B = 2000
NB = N_NODES // B

_mesh = plsc.VectorSubcoreMesh(
    core_axis_name="c", subcore_axis_name="s", num_cores=NC, num_subcores=NS)


def _f32(shape):
  return jax.ShapeDtypeStruct(shape, jnp.float32)


def _edge_pipeline(ngroups, tile_c0, gstride, src_hbm, dst_hbm, tbl_hbm,
                   idxs, idxd, rows, acc, semst, semg, sems):
  gather = tbl_hbm is not None
  lag = DEPTH - 1

  def stage(k, b):
    c0 = tile_c0 + k * gstride
    if gather:
      pltpu.async_copy(src_hbm.at[pl.ds(c0, STG)], idxs.at[b], semst)
    pltpu.async_copy(dst_hbm.at[pl.ds(c0, STG)], idxd.at[b], semst)

  def drain_stage(b):
    if gather:
      pltpu.make_async_copy(src_hbm.at[pl.ds(0, STG)], idxs.at[b],
                            semst).wait()
    pltpu.make_async_copy(dst_hbm.at[pl.ds(0, STG)], idxd.at[b],
                          semst).wait()

  def drain_scatters(b):
    def drain_s(j, c2):
      if gather:
        pltpu.make_async_copy(rows.at[b, j], acc.at[idxd.at[b, j]],
                              sems).wait()
      else:
        pltpu.make_async_copy(rows, acc.at[idxd.at[b, j]], sems).wait()
      return c2

    lax.fori_loop(0, STG, drain_s, 0)

  stage(0, 0)

  def group_body(k, carry):
    b = lax.rem(k, DEPTH)

    @pl.when(k >= lag)
    def _():
      drain_scatters(lax.rem(k + 1, DEPTH))

    drain_stage(b)

    @pl.when(k + 1 < ngroups)
    def _():
      stage(k + 1, lax.rem(k + 1, DEPTH))

    if gather:
      def fire_g(j, c2):
        pltpu.async_copy(tbl_hbm.at[idxs.at[b, j]], rows.at[b, j], semg)
        return c2

      lax.fori_loop(0, STG, fire_g, 0)

      def drain_g(j, c2):
        pltpu.make_async_copy(tbl_hbm.at[idxs.at[b, j]], rows.at[b, j],
                              semg).wait()
        return c2

      lax.fori_loop(0, STG, drain_g, 0)

    def fire_s(j, c2):
      if gather:
        pltpu.async_copy(rows.at[b, j], acc.at[idxd.at[b, j]], sems,
                         add=True)
      else:
        pltpu.async_copy(rows, acc.at[idxd.at[b, j]], sems, add=True)
      return c2

    lax.fori_loop(0, STG, fire_s, 0)
    return carry

  lax.fori_loop(0, ngroups, group_body, 0)

  def drain_tail(k, carry):
    drain_scatters(lax.rem(k, DEPTH))
    return carry

  lax.fori_loop(jnp.maximum(ngroups - lag, 0), ngroups, drain_tail, 0)


@functools.partial(
    pl.kernel,
    out_type=[_f32((N_PAD, 16)), _f32((N_PAD, 16))],
    mesh=_mesh,
    compiler_params=pltpu.CompilerParams(use_tc_tiling_on_sc=False),
    scratch_types=[
        pltpu.VMEM((DEPTH, STG, CHUNK), jnp.int32),
        pltpu.VMEM((CHUNK, 16), jnp.float32),
        pltpu.VMEM_SHARED((N_PAD, 16), jnp.float32),
        pltpu.SemaphoreType.DMA,
        pltpu.SemaphoreType.DMA,
    ],
)
def _deg_kernel(dst_hbm, ones_hbm, zeros_hbm, out0, out1,
                idxd, ones_v, acc, semst, sems):
  c = lax.axis_index("c")
  s = lax.axis_index("s")
  w = c * NS + s

  pltpu.sync_copy(ones_hbm, ones_v)
  row0 = s * ROWS_PER_TILE
  pltpu.sync_copy(zeros_hbm.at[pl.ds(row0, ROWS_PER_TILE)],
                  acc.at[pl.ds(row0, ROWS_PER_TILE)])
  plsc.subcore_barrier()

  ngroups = NCHUNKS // (NC * NS * STG)
  _edge_pipeline(ngroups, s * (NCHUNKS // NS) + c * STG, 2 * STG,
                 None, dst_hbm, None, None, idxd, ones_v, acc,
                 semst, None, sems)
  plsc.subcore_barrier()

  @pl.when(c == 0)
  def _():
    pltpu.sync_copy(acc.at[pl.ds(row0, ROWS_PER_TILE)],
                    out0.at[pl.ds(row0, ROWS_PER_TILE)])

  @pl.when(c == 1)
  def _():
    pltpu.sync_copy(acc.at[pl.ds(row0, ROWS_PER_TILE)],
                    out1.at[pl.ds(row0, ROWS_PER_TILE)])


@functools.partial(
    pl.kernel,
    out_type=[_f32((N_PAD, 16)), _f32((N_PAD, 16))],
    mesh=_mesh,
    compiler_params=pltpu.CompilerParams(use_tc_tiling_on_sc=False),
    scratch_types=[
        pltpu.VMEM((DEPTH, STG, CHUNK), jnp.int32),
        pltpu.VMEM((DEPTH, STG, CHUNK), jnp.int32),
        pltpu.VMEM((DEPTH, STG, CHUNK, 16), jnp.float32),
        pltpu.VMEM_SHARED((N_PAD, 16), jnp.float32),
        pltpu.SemaphoreType.DMA,
        pltpu.SemaphoreType.DMA,
        pltpu.SemaphoreType.DMA,
    ],
)
def _agg1_kernel(src_hbm, dst_hbm, tbl_hbm, zeros_hbm, out0, out1,
                 idxs, idxd, rows, acc, semst, semg, sems):
  c = lax.axis_index("c")
  s = lax.axis_index("s")

  row0 = s * ROWS_PER_TILE
  pltpu.sync_copy(zeros_hbm.at[pl.ds(row0, ROWS_PER_TILE)],
                  acc.at[pl.ds(row0, ROWS_PER_TILE)])
  plsc.subcore_barrier()

  ngroups = NCHUNKS // (NC * NS * STG)
  _edge_pipeline(ngroups, s * (NCHUNKS // NS) + c * STG, 2 * STG,
                 src_hbm, dst_hbm, tbl_hbm, idxs, idxd, rows, acc,
                 semst, semg, sems)
  plsc.subcore_barrier()

  @pl.when(c == 0)
  def _():
    pltpu.sync_copy(acc.at[pl.ds(row0, ROWS_PER_TILE)],
                    out0.at[pl.ds(row0, ROWS_PER_TILE)])

  @pl.when(c == 1)
  def _():
    pltpu.sync_copy(acc.at[pl.ds(row0, ROWS_PER_TILE)],
                    out1.at[pl.ds(row0, ROWS_PER_TILE)])


@functools.partial(
    pl.kernel,
    out_type=[_f32((N_PAD, 16))] * 4,
    mesh=_mesh,
    compiler_params=pltpu.CompilerParams(use_tc_tiling_on_sc=False),
    scratch_types=[
        pltpu.VMEM((DEPTH, STG, CHUNK), jnp.int32),
        pltpu.VMEM((DEPTH, STG, CHUNK), jnp.int32),
        pltpu.VMEM((DEPTH, STG, CHUNK, 16), jnp.float32),
        pltpu.VMEM_SHARED((N_PAD, 16), jnp.float32),
        pltpu.SemaphoreType.DMA,
        pltpu.SemaphoreType.DMA,
        pltpu.SemaphoreType.DMA,
    ],
)
def _agg2_kernel(src_hbm, dst_hbm, h0, h1, h2, h3, zeros_hbm,
                 o0, o1, o2, o3, idxs, idxd, rows, acc, semst, semg, sems):
  c = lax.axis_index("c")
  s = lax.axis_index("s")
  tbls = [h0, h1, h2, h3]
  outs = [o0, o1, o2, o3]

  row0 = s * ROWS_PER_TILE
  chunks_per_tile = NCHUNKS // NS

  for g in range(4):
    active = c == (g // 2)

    @pl.when(active)
    def _(g=g):
      pltpu.sync_copy(zeros_hbm.at[pl.ds(row0, ROWS_PER_TILE)],
                      acc.at[pl.ds(row0, ROWS_PER_TILE)])

    plsc.subcore_barrier()

    @pl.when(active)
    def _(g=g):
      _edge_pipeline(chunks_per_tile // STG, s * chunks_per_tile, STG,
                     src_hbm, dst_hbm, tbls[g], idxs, idxd, rows, acc,
                     semst, semg, sems)

    plsc.subcore_barrier()

    @pl.when(active)
    def _(g=g):
      pltpu.sync_copy(acc.at[pl.ds(row0, ROWS_PER_TILE)],
                      outs[g].at[pl.ds(row0, ROWS_PER_TILE)])

    plsc.subcore_barrier()


def _scale_body(x_ref, d0_ref, d1_ref, p_ref, xs_ref, dinv_ref):
  deg = d0_ref[:, 0:1] + d1_ref[:, 0:1] + 1.0
  dinv = lax.rsqrt(deg)
  xs_ref[...] = (x_ref[...] * dinv) @ p_ref[...]
  dinv_ref[...] = dinv


def _scale_call(x, degp0, degp1, pmat):
  return pl.pallas_call(
      _scale_body,
      grid=(NB,),
      in_specs=[
          pl.BlockSpec((B, IN_CH), lambda i: (i, 0)),
          pl.BlockSpec((B, 16), lambda i: (i, 0)),
          pl.BlockSpec((B, 16), lambda i: (i, 0)),
          pl.BlockSpec((IN_CH, 16), lambda i: (0, 0)),
      ],
      out_specs=[
          pl.BlockSpec((B, 16), lambda i: (i, 0)),
          pl.BlockSpec((B, 1), lambda i: (i, 0)),
      ],
      out_shape=[_f32((N_PAD, 16)), _f32((N_NODES, 1))],
  )(x, degp0, degp1, pmat)


def _h1_body(p0_ref, p1_ref, xs_ref, dinv_ref, w1_ref, b1_ref,
             h0_ref, h1_ref, h2_ref, h3_ref):
  dinv = dinv_ref[...]
  t = (p0_ref[...] + p1_ref[...] + xs_ref[...]) * dinv
  outs = [h0_ref, h1_ref, h2_ref, h3_ref]
  for g in range(4):
    hg = jnp.maximum(t @ w1_ref[g] + b1_ref[g], 0.0)
    outs[g][...] = hg * dinv


def _h1_call(p0, p1, xs16, dinv, w1s, b1r):
  return pl.pallas_call(
      _h1_body,
      grid=(NB,),
      in_specs=[
          pl.BlockSpec((B, 16), lambda i: (i, 0)),
          pl.BlockSpec((B, 16), lambda i: (i, 0)),
          pl.BlockSpec((B, 16), lambda i: (i, 0)),
          pl.BlockSpec((B, 1), lambda i: (i, 0)),
          pl.BlockSpec((4, 16, 16), lambda i: (0, 0, 0)),
          pl.BlockSpec((4, 16), lambda i: (0, 0)),
      ],
      out_specs=[pl.BlockSpec((B, 16), lambda i: (i, 0))] * 4,
      out_shape=[_f32((N_PAD, 16))] * 4,
  )(p0, p1, xs16, dinv, w1s, b1r)


def _pool_body(o0_ref, o1_ref, o2_ref, o3_ref,
               s0_ref, s1_ref, s2_ref, s3_ref,
               dinv_ref, batch_ref, b2_ref, wfc_ref, bfc_ref,
               out_ref, sums_ref, cnt_ref):
  i = pl.program_id(0)

  @pl.when(i == 0)
  def _():
    sums_ref[...] = jnp.zeros((4, N_GRAPHS, 16), jnp.float32)
    cnt_ref[...] = jnp.zeros((N_GRAPHS, 1), jnp.float32)

  dinv = dinv_ref[...]
  bb = batch_ref[0]
  oht = (lax.broadcasted_iota(jnp.int32, (N_GRAPHS, B), 0) == bb
         ).astype(jnp.float32)
  cnt_ref[...] += lax.dot_general(
      oht, jnp.ones((B, 1), jnp.float32), (((1,), (0,)), ((), ())),
      preferred_element_type=jnp.float32)
  os_ = [o0_ref, o1_ref, o2_ref, o3_ref]
  ss_ = [s0_ref, s1_ref, s2_ref, s3_ref]
  for g in range(4):
    h2g = jnp.maximum((os_[g][...] + ss_[g][...]) * dinv + b2_ref[g], 0.0)
    sums_ref[g] += lax.dot_general(
        oht, h2g, (((1,), (0,)), ((), ())),
        preferred_element_type=jnp.float32)

  @pl.when(i == NB - 1)
  def _():
    cnt = jnp.maximum(cnt_ref[...], 1.0)
    logits = bfc_ref[...]
    for g in range(4):
      logits = logits + lax.dot_general(
          sums_ref[g] / cnt, wfc_ref[g], (((1,), (0,)), ((), ())),
          preferred_element_type=jnp.float32)
    out_ref[...] = 1.0 / (1.0 + jnp.exp(-logits))


def _pool_call(o_parts, h1s_parts, dinv, batch2d, b2r, wfcs, bfc2d):
  return pl.pallas_call(
      _pool_body,
      grid=(NB,),
      in_specs=(
          [pl.BlockSpec((B, 16), lambda i: (i, 0))] * 8 + [
              pl.BlockSpec((B, 1), lambda i: (i, 0)),
              pl.BlockSpec((1, 1, B), lambda i: (i, 0, 0)),
              pl.BlockSpec((4, 16), lambda i: (0, 0)),
              pl.BlockSpec((4, 16, OUT_CH), lambda i: (0, 0, 0)),
              pl.BlockSpec((1, OUT_CH), lambda i: (0, 0)),
          ]),
      out_specs=pl.BlockSpec((N_GRAPHS, OUT_CH), lambda i: (0, 0)),
      out_shape=_f32((N_GRAPHS, OUT_CH)),
      scratch_shapes=[
          pltpu.VMEM((4, N_GRAPHS, 16), jnp.float32),
          pltpu.VMEM((N_GRAPHS, 1), jnp.float32),
      ],
  )(*o_parts, *h1s_parts, dinv, batch2d, b2r, wfcs, bfc2d)


def kernel(x, edge_index, batch, W1, b1, W2, b2, Wfc, bfc):
  f32 = jnp.float32
  npad_e = E_PAD - N_EDGES
  dummy = jnp.full((npad_e,), N_NODES, jnp.int32)
  src2d = jnp.concatenate([edge_index[0], dummy]).reshape(NCHUNKS, CHUNK)
  dst2d = jnp.concatenate([edge_index[1], dummy]).reshape(NCHUNKS, CHUNK)

  ones_rows = jnp.ones((CHUNK, 16), f32)
  zeros_tbl = jnp.zeros((N_PAD, 16), f32)
  pmat = jnp.eye(IN_CH, 16, dtype=f32)

  w1p = jnp.concatenate([W1, jnp.zeros((16 - IN_CH, HID), f32)], axis=0)
  w1s = jnp.stack([w1p[:, 16 * g:16 * (g + 1)] for g in range(4)])
  b1r = b1.reshape(4, 16)
  b2r = b2.reshape(4, 16)
  wfcs = jnp.stack([Wfc[16 * g:16 * (g + 1), :] for g in range(4)])
  bfc2d = bfc.reshape(1, OUT_CH)
  batch2d = batch.reshape(NB, 1, B)

  degp0, degp1 = _deg_kernel(dst2d, ones_rows, zeros_tbl)
  xs16, dinv = _scale_call(x, degp0, degp1, pmat)
  p0, p1 = _agg1_kernel(src2d, dst2d, xs16, zeros_tbl)
  h1s = _h1_call(p0, p1, xs16, dinv, w1s, b1r)
  o_parts = _agg2_kernel(src2d, dst2d, *h1s, zeros_tbl)
  return _pool_call(o_parts, h1s, dinv, batch2d, b2r, wfcs, bfc2d)
